# baseline (speedup 1.0000x reference)
import functools

import jax
import jax.numpy as jnp
from jax import lax
from jax.experimental import pallas as pl
from jax.experimental.pallas import tpu as pltpu
from jax.experimental.pallas import tpu_sc as plsc

N = 10000
E = 320000
K = 128
EB = E // K
D_IN, D_HID, D_OUT = 128, 64, 128
RPT = N // 16
RCH = 125

C0B = 107
B0 = 16 * C0B
C1BA = 50
C1BB = 49
assert B0 + 4 * C1BA + 12 * C1BB == EB

_mesh = plsc.VectorSubcoreMesh(core_axis_name="c", subcore_axis_name="s")
_sc_params = pltpu.CompilerParams(use_tc_tiling_on_sc=False)


@functools.partial(
    pl.kernel,
    mesh=_mesh,
    out_type=jax.ShapeDtypeStruct((2, N, 16), jnp.float32),
    scratch_types=[
        pltpu.VMEM((79, 2, K), jnp.int32),
        pltpu.VMEM((K, 16), jnp.float32),
        pltpu.VMEM((RCH, 16), jnp.float32),
        pltpu.VMEM_SHARED((N, 16), jnp.float32),
    ],
    compiler_params=_sc_params,
)
def _deg(ei_hbm, out_hbm, ei_v, ones_v, zb_v, acc):
    c = lax.axis_index("c")
    s = lax.axis_index("s")

    @pl.loop(0, K)
    def _fill(i):
        ones_v[i, :] = jnp.ones((16,), jnp.float32)

    @pl.loop(0, RCH)
    def _fillz(i):
        zb_v[i, :] = jnp.zeros((16,), jnp.float32)

    @pl.loop(0, RPT // RCH)
    def _zero(k):
        pltpu.sync_copy(zb_v, acc.at[pl.ds(s * RPT + k * RCH, RCH)])

    plsc.subcore_barrier()

    def _hist(nblk, base):
        pltpu.sync_copy(ei_hbm.at[pl.ds(base, nblk)],
                        ei_v.at[pl.ds(0, nblk)])

        @pl.loop(0, nblk)
        def _blk(j):
            pltpu.sync_copy(ones_v, acc.at[ei_v.at[j, 1]], add=True)

    @pl.when(s < 2)
    def _sa():
        _hist(79, c * 1250 + s * 79)

    @pl.when(s >= 2)
    def _sb():
        _hist(78, c * 1250 + 158 + (s - 2) * 78)

    plsc.subcore_barrier()

    @pl.loop(0, RPT // RCH)
    def _out(k):
        start = s * RPT + k * RCH
        pltpu.sync_copy(acc.at[pl.ds(start, RCH)],
                        out_hbm.at[c, pl.ds(start, RCH)])


@functools.partial(
    pl.kernel,
    mesh=_mesh,
    out_type=jax.ShapeDtypeStruct((2, N, D_HID), jnp.float32),
    scratch_types=[
        pltpu.VMEM((C0B, 2, K), jnp.int32),
        pltpu.VMEM((K, D_HID), jnp.float32),
        pltpu.VMEM((RCH, D_HID), jnp.float32),
        pltpu.VMEM_SHARED((N, D_HID), jnp.float32),
        pltpu.SemaphoreType.DMA,
    ],
    compiler_params=_sc_params,
)
def _agg(g_hbm, ei_hbm, out_hbm, ei_v, rows_v, zb_v, acc, sem):
    c = lax.axis_index("c")
    s = lax.axis_index("s")

    @pl.loop(0, RCH)
    def _fill(i):
        @pl.loop(0, D_HID, step=16)
        def _fill16(j):
            zb_v[i, pl.ds(j, 16)] = jnp.zeros((16,), jnp.float32)

    @pl.loop(0, RPT // RCH)
    def _zero(k):
        pltpu.sync_copy(zb_v, acc.at[pl.ds(s * RPT + k * RCH, RCH)])

    plsc.subcore_barrier()

    def _run_edges(nblk, base):
        pltpu.sync_copy(ei_hbm.at[pl.ds(base, nblk)],
                        ei_v.at[pl.ds(0, nblk)])

        @pl.loop(0, nblk)
        def _blk(j):
            pltpu.async_copy(g_hbm.at[ei_v.at[j, 0]], rows_v, sem).wait()
            pltpu.sync_copy(rows_v, acc.at[ei_v.at[j, 1]], add=True)

    @pl.when(c == 0)
    def _core0():
        _run_edges(C0B, s * C0B)

    @pl.when(jnp.logical_and(c == 1, s < 4))
    def _core1a():
        _run_edges(C1BA, B0 + s * C1BA)

    @pl.when(jnp.logical_and(c == 1, s >= 4))
    def _core1b():
        _run_edges(C1BB, B0 + 4 * C1BA + (s - 4) * C1BB)

    plsc.subcore_barrier()

    @pl.loop(0, RPT // RCH)
    def _out(k):
        start = s * RPT + k * RCH
        pltpu.sync_copy(acc.at[pl.ds(start, RCH)],
                        out_hbm.at[c, pl.ds(start, RCH)])


def _dinv_of(deg_ref):
    cnt = deg_ref[0, :, 0:1] + deg_ref[1, :, 0:1]
    return lax.rsqrt(cnt + 1.0)


def _mm1_body(deg_ref, x_ref, w_ref, o_ref):
    h = lax.dot(x_ref[...], w_ref[...], precision=lax.Precision.HIGHEST)
    o_ref[...] = _dinv_of(deg_ref) * h


_mm1 = pl.pallas_call(
    _mm1_body,
    out_shape=jax.ShapeDtypeStruct((N, D_HID), jnp.float32),
)


def _mid_body(p_ref, g_ref, deg_ref, b_ref, o_ref):
    dinv = _dinv_of(deg_ref)
    a = dinv * (p_ref[0] + p_ref[1] + g_ref[...]) + b_ref[...]
    o_ref[...] = dinv * jnp.maximum(a, 0.0)


_mid = pl.pallas_call(
    _mid_body,
    out_shape=jax.ShapeDtypeStruct((N, D_HID), jnp.float32),
)


def _final_body(p_ref, g_ref, deg_ref, w_ref, b_ref, o_ref):
    dinv = _dinv_of(deg_ref)
    a = dinv * (p_ref[0] + p_ref[1] + g_ref[...])
    o_ref[...] = lax.dot(a, w_ref[...],
                         precision=lax.Precision.HIGHEST) + b_ref[...]


_final = pl.pallas_call(
    _final_body,
    out_shape=jax.ShapeDtypeStruct((N, D_OUT), jnp.float32),
)


@jax.jit
def kernel(x, edge_index, W1, b1, W2, b2):
    ei3 = edge_index.reshape(2, EB, K).transpose(1, 0, 2)

    deg = _deg(ei3)
    g1 = _mm1(deg, x, W1)
    p1 = _agg(g1, ei3)
    g2 = _mid(p1, g1, deg, b1.reshape(1, D_HID))
    p2 = _agg(g2, ei3)
    out = _final(p2, g2, deg, W2, b2.reshape(1, D_OUT))
    return out


# speedup vs baseline: 1.2062x; 1.2062x over previous
import functools

import jax
import jax.numpy as jnp
from jax import lax
from jax.experimental import pallas as pl
from jax.experimental.pallas import tpu as pltpu
from jax.experimental.pallas import tpu_sc as plsc

N = 10000
E = 320000
K = 128
EB = E // K
D_IN, D_HID, D_OUT = 128, 64, 128
RPT = N // 16
RCH = 125

C0B = 80
B0 = 16 * C0B
C1BA = 77
C1BB = 76
assert B0 + 4 * C1BA + 12 * C1BB == EB

_mesh = plsc.VectorSubcoreMesh(core_axis_name="c", subcore_axis_name="s")
_sc_params = pltpu.CompilerParams(use_tc_tiling_on_sc=False)


@functools.partial(
    pl.kernel,
    mesh=_mesh,
    out_type=jax.ShapeDtypeStruct((2, N, 16), jnp.float32),
    scratch_types=[
        pltpu.VMEM((79, 2, K), jnp.int32),
        pltpu.VMEM((K, 16), jnp.float32),
        pltpu.VMEM((RCH, 16), jnp.float32),
        pltpu.VMEM_SHARED((N, 16), jnp.float32),
    ],
    compiler_params=_sc_params,
)
def _deg(ei_hbm, out_hbm, ei_v, ones_v, zb_v, acc):
    c = lax.axis_index("c")
    s = lax.axis_index("s")

    @pl.loop(0, K)
    def _fill(i):
        ones_v[i, :] = jnp.ones((16,), jnp.float32)

    @pl.loop(0, RCH)
    def _fillz(i):
        zb_v[i, :] = jnp.zeros((16,), jnp.float32)

    @pl.loop(0, RPT // RCH)
    def _zero(k):
        pltpu.sync_copy(zb_v, acc.at[pl.ds(s * RPT + k * RCH, RCH)])

    plsc.subcore_barrier()

    def _hist(nblk, base):
        pltpu.sync_copy(ei_hbm.at[pl.ds(base, nblk)],
                        ei_v.at[pl.ds(0, nblk)])

        @pl.loop(0, nblk)
        def _blk(j):
            pltpu.sync_copy(ones_v, acc.at[ei_v.at[j, 1]], add=True)

    @pl.when(s < 2)
    def _sa():
        _hist(79, c * 1250 + s * 79)

    @pl.when(s >= 2)
    def _sb():
        _hist(78, c * 1250 + 158 + (s - 2) * 78)

    plsc.subcore_barrier()

    @pl.loop(0, RPT // RCH)
    def _out(k):
        start = s * RPT + k * RCH
        pltpu.sync_copy(acc.at[pl.ds(start, RCH)],
                        out_hbm.at[c, pl.ds(start, RCH)])


@functools.partial(
    pl.kernel,
    mesh=_mesh,
    out_type=jax.ShapeDtypeStruct((2, N, D_HID), jnp.float32),
    scratch_types=[
        pltpu.VMEM((C0B, 2, K), jnp.int32),
        pltpu.VMEM((K, D_HID), jnp.float32),
        pltpu.VMEM((RCH, D_HID), jnp.float32),
        pltpu.VMEM_SHARED((N, D_HID), jnp.float32),
        pltpu.SemaphoreType.DMA,
    ],
    compiler_params=_sc_params,
)
def _agg(g_hbm, ei_hbm, out_hbm, ei_v, rows_v, zb_v, acc, sem):
    c = lax.axis_index("c")
    s = lax.axis_index("s")

    @pl.loop(0, RCH)
    def _fill(i):
        @pl.loop(0, D_HID, step=16)
        def _fill16(j):
            zb_v[i, pl.ds(j, 16)] = jnp.zeros((16,), jnp.float32)

    @pl.loop(0, RPT // RCH)
    def _zero(k):
        pltpu.sync_copy(zb_v, acc.at[pl.ds(s * RPT + k * RCH, RCH)])

    plsc.subcore_barrier()

    def _run_edges(nblk, base):
        pltpu.sync_copy(ei_hbm.at[pl.ds(base, nblk)],
                        ei_v.at[pl.ds(0, nblk)])

        @pl.loop(0, nblk)
        def _blk(j):
            pltpu.async_copy(g_hbm.at[ei_v.at[j, 0]], rows_v, sem).wait()
            pltpu.sync_copy(rows_v, acc.at[ei_v.at[j, 1]], add=True)

    @pl.when(c == 0)
    def _core0():
        _run_edges(C0B, s * C0B)

    @pl.when(jnp.logical_and(c == 1, s < 4))
    def _core1a():
        _run_edges(C1BA, B0 + s * C1BA)

    @pl.when(jnp.logical_and(c == 1, s >= 4))
    def _core1b():
        _run_edges(C1BB, B0 + 4 * C1BA + (s - 4) * C1BB)

    plsc.subcore_barrier()

    @pl.loop(0, RPT // RCH)
    def _out(k):
        start = s * RPT + k * RCH
        pltpu.sync_copy(acc.at[pl.ds(start, RCH)],
                        out_hbm.at[c, pl.ds(start, RCH)])


def _dinv_of(deg_ref):
    cnt = deg_ref[0, :, 0:1] + deg_ref[1, :, 0:1]
    return lax.rsqrt(cnt + 1.0)


def _mm1_body(deg_ref, x_ref, w_ref, o_ref):
    h = lax.dot(x_ref[...], w_ref[...], precision=lax.Precision.HIGHEST)
    o_ref[...] = _dinv_of(deg_ref) * h


_mm1 = pl.pallas_call(
    _mm1_body,
    out_shape=jax.ShapeDtypeStruct((N, D_HID), jnp.float32),
)


def _mid_body(p_ref, g_ref, deg_ref, b_ref, o_ref):
    dinv = _dinv_of(deg_ref)
    a = dinv * (p_ref[0] + p_ref[1] + g_ref[...]) + b_ref[...]
    o_ref[...] = dinv * jnp.maximum(a, 0.0)


_mid = pl.pallas_call(
    _mid_body,
    out_shape=jax.ShapeDtypeStruct((N, D_HID), jnp.float32),
)


def _final_body(p_ref, g_ref, deg_ref, w_ref, b_ref, o_ref):
    dinv = _dinv_of(deg_ref)
    a = dinv * (p_ref[0] + p_ref[1] + g_ref[...])
    o_ref[...] = lax.dot(a, w_ref[...],
                         precision=lax.Precision.HIGHEST) + b_ref[...]


_final = pl.pallas_call(
    _final_body,
    out_shape=jax.ShapeDtypeStruct((N, D_OUT), jnp.float32),
)


@jax.jit
def kernel(x, edge_index, W1, b1, W2, b2):
    ei3 = edge_index.reshape(2, EB, K).transpose(1, 0, 2)

    deg = _deg(ei3)
    g1 = _mm1(deg, x, W1)
    p1 = _agg(g1, ei3)
    g2 = _mid(p1, g1, deg, b1.reshape(1, D_HID))
    p2 = _agg(g2, ei3)
    out = _final(p2, g2, deg, W2, b2.reshape(1, D_OUT))
    return out
